# baseline (speedup 1.0000x reference)
import jax
import jax.numpy as jnp
from jax import lax
from jax.experimental import pallas as pl
from jax.experimental.pallas import tpu as pltpu
from jax.experimental.pallas import tpu_sc as plsc

B, C, HP, WP = 4, 96, 112, 112
HO, WO = 224, 224
NQ = B * C * 4
IN_Q = HP * WP * 2 // 4
OUT_Q = HO * WO * 2 // 4
NC, NS, L = 2, 16, 16
NW = NC * NS
QPW = NQ // NW
UNROLL = 8


def _unpool_body(zf, idxf, off_hbm, outf,
                 v0, v1, i0, i1, ob0, ob1, offv,
                 sin0, sin1, sout0, sout1):
    wid = lax.axis_index("s") * NC + lax.axis_index("c")
    pltpu.sync_copy(off_hbm, offv)
    off = offv[...]
    lane = lax.iota(jnp.int32, L)
    comp = lane & 1
    zeros = jnp.zeros((L,), jnp.float32)
    q0 = wid * QPW
    vbufs, ibufs, obufs = (v0, v1), (i0, i1), (ob0, ob1)
    sins, souts = (sin0, sin1), (sout0, sout1)

    def issue_in(q, par):
        pltpu.async_copy(zf.at[q], vbufs[par], sins[par])
        pltpu.async_copy(idxf.at[q], ibufs[par], sins[par])

    def wait_in(q, par):
        pltpu.make_async_copy(zf.at[q], vbufs[par], sins[par]).wait()
        pltpu.make_async_copy(idxf.at[q], ibufs[par], sins[par]).wait()

    issue_in(q0, 0)

    def gbody(g, carry):
        for par in range(2):
            step = g * 2 + par
            q = q0 + step

            @pl.when(step + 1 < QPW)
            def _():
                issue_in(q + 1, 1 - par)

            @pl.when(step >= 2)
            def _():
                pltpu.make_async_copy(
                    obufs[par], outf.at[q - 2], souts[par]
                ).wait()

            ob = obufs[par]

            @plsc.parallel_loop(0, OUT_Q, step=L, unroll=UNROLL)
            def _(t):
                ob[pl.ds(t, L)] = zeros

            wait_in(q, par)
            basew = (q & 3) * OUT_Q
            vv, ii = vbufs[par], ibufs[par]

            @plsc.parallel_loop(0, IN_Q, step=L, unroll=UNROLL)
            def _(t):
                v = vv[pl.ds(t, L)]
                ix = ii[pl.ds(t, L)]
                local = (ix + off) * 2 + comp - basew
                plsc.store_scatter(ob, [local], v)

            pltpu.async_copy(ob, outf.at[q], souts[par])
        return carry

    lax.fori_loop(0, QPW // 2, gbody, 0)
    for par in range(2):
        q = q0 + QPW - 2 + par
        pltpu.make_async_copy(obufs[par], outf.at[q], souts[par]).wait()


def kernel(z, indices, output_size):
    zf = z.reshape(NQ, IN_Q)
    idxf = indices.reshape(NQ, IN_Q)
    off = jnp.broadcast_to(jnp.asarray(output_size, jnp.int32) - HO, (L,))
    mesh = plsc.VectorSubcoreMesh(core_axis_name="c", subcore_axis_name="s")
    out = pl.kernel(
        _unpool_body,
        out_type=jax.ShapeDtypeStruct((NQ, OUT_Q), jnp.float32),
        mesh=mesh,
        compiler_params=pltpu.CompilerParams(needs_layout_passes=False),
        scratch_types=[
            pltpu.VMEM((IN_Q,), jnp.float32),
            pltpu.VMEM((IN_Q,), jnp.float32),
            pltpu.VMEM((IN_Q,), jnp.int32),
            pltpu.VMEM((IN_Q,), jnp.int32),
            pltpu.VMEM((OUT_Q,), jnp.float32),
            pltpu.VMEM((OUT_Q,), jnp.float32),
            pltpu.VMEM((L,), jnp.int32),
            pltpu.SemaphoreType.DMA,
            pltpu.SemaphoreType.DMA,
            pltpu.SemaphoreType.DMA,
            pltpu.SemaphoreType.DMA,
        ],
    )(zf, idxf, off)
    return out.reshape(B, C, HO, WO, 2)


# speedup vs baseline: 30.9748x; 30.9748x over previous
import jax
import jax.numpy as jnp
from jax import lax
from jax.experimental import pallas as pl
from jax.experimental.pallas import tpu as pltpu
from jax.experimental.pallas import tpu_sc as plsc

B, C, HP, WP = 4, 96, 112, 112
HO, WO = 224, 224
PLANES = B * C
IN_P = HP * WP * 2
OUT_P = HO * WO * 2
NQ = PLANES * 4
IN_Q = IN_P // 4
OUT_Q = OUT_P // 4
NC, NS, L = 2, 16, 16
NW = NC * NS
QPW = NQ // NW
UNROLL = 8


def _unpool_body(zf, idxf, off_hbm, outf,
                 v0, v1, i0, i1, ob0, ob1, offv,
                 sin0, sin1, sout0, sout1):
    wid = lax.axis_index("s") * NC + lax.axis_index("c")
    pltpu.sync_copy(off_hbm, offv)
    off = offv[...]
    lane = lax.iota(jnp.int32, L)
    comp = lane & 1
    zeros = jnp.zeros((L,), jnp.float32)
    q0 = wid * QPW
    vbufs, ibufs, obufs = (v0, v1), (i0, i1), (ob0, ob1)
    sins, souts = (sin0, sin1), (sout0, sout1)

    def odst(q):
        return outf.at[q // 4, pl.ds((q & 3) * OUT_Q, OUT_Q)]

    def issue_in(q, par):
        src_v = zf.at[q // 4, pl.ds((q & 3) * IN_Q, IN_Q)]
        src_i = idxf.at[q // 4, pl.ds((q & 3) * IN_Q, IN_Q)]
        pltpu.async_copy(src_v, vbufs[par], sins[par])
        pltpu.async_copy(src_i, ibufs[par], sins[par])

    def wait_in(q, par):
        src_v = zf.at[q // 4, pl.ds((q & 3) * IN_Q, IN_Q)]
        src_i = idxf.at[q // 4, pl.ds((q & 3) * IN_Q, IN_Q)]
        pltpu.make_async_copy(src_v, vbufs[par], sins[par]).wait()
        pltpu.make_async_copy(src_i, ibufs[par], sins[par]).wait()

    issue_in(q0, 0)

    def gbody(g, carry):
        for par in range(2):
            step = g * 2 + par
            q = q0 + step

            @pl.when(step + 1 < QPW)
            def _():
                issue_in(q + 1, 1 - par)

            @pl.when(step >= 2)
            def _():
                pltpu.make_async_copy(obufs[par], odst(q - 2), souts[par]).wait()

            ob = obufs[par]

            @plsc.parallel_loop(0, OUT_Q, step=L, unroll=UNROLL)
            def _(t):
                ob[pl.ds(t, L)] = zeros

            wait_in(q, par)
            basew = (q & 3) * OUT_Q
            vv, ii = vbufs[par], ibufs[par]

            @plsc.parallel_loop(0, IN_Q, step=L, unroll=UNROLL)
            def _(t):
                v = vv[pl.ds(t, L)]
                ix = ii[pl.ds(t, L)]
                local = (ix + off) * 2 + comp - basew
                plsc.store_scatter(ob, [local], v)

            pltpu.async_copy(ob, odst(q), souts[par])
        return carry

    lax.fori_loop(0, QPW // 2, gbody, 0)
    for par in range(2):
        q = q0 + QPW - 2 + par
        pltpu.make_async_copy(obufs[par], odst(q), souts[par]).wait()


def kernel(z, indices, output_size):
    zf = z.reshape(PLANES, IN_P)
    idxf = indices.reshape(PLANES, IN_P)
    off = jnp.broadcast_to(jnp.asarray(output_size, jnp.int32) - HO, (L,))
    mesh = plsc.VectorSubcoreMesh(core_axis_name="c", subcore_axis_name="s")
    out = pl.kernel(
        _unpool_body,
        out_type=jax.ShapeDtypeStruct((PLANES, OUT_P), jnp.float32),
        mesh=mesh,
        compiler_params=pltpu.CompilerParams(needs_layout_passes=False),
        scratch_types=[
            pltpu.VMEM((IN_Q,), jnp.float32),
            pltpu.VMEM((IN_Q,), jnp.float32),
            pltpu.VMEM((IN_Q,), jnp.int32),
            pltpu.VMEM((IN_Q,), jnp.int32),
            pltpu.VMEM((OUT_Q,), jnp.float32),
            pltpu.VMEM((OUT_Q,), jnp.float32),
            pltpu.VMEM((L,), jnp.int32),
            pltpu.SemaphoreType.DMA,
            pltpu.SemaphoreType.DMA,
            pltpu.SemaphoreType.DMA,
            pltpu.SemaphoreType.DMA,
        ],
    )(zf, idxf, off)
    return out.reshape(B, C, HO, WO, 2)
